# baseline (speedup 1.0000x reference)
import functools

import jax
import jax.numpy as jnp
from jax import lax
from jax.experimental import pallas as pl
from jax.experimental.pallas import tpu as pltpu
from jax.experimental.pallas import tpu_sc as plsc

BATCH = 16384
DIM = 128
LANES = 16
NUM_CORES = 2
NUM_SUBCORES = 16
NUM_WORKERS = NUM_CORES * NUM_SUBCORES
ROWS_PER_WORKER = BATCH // NUM_WORKERS
CHUNK = 128
NUM_CHUNKS = ROWS_PER_WORKER // CHUNK
VPR = DIM // LANES


def _sc_partial_body(data_hbm, labels_hbm, cen_hbm, out_hbm,
                     idx_v, d0, d1, c0, c1, acc_v,
                     sem_i, sem_d0, sem_d1, sem_c0, sem_c1):
  wid = lax.axis_index("s") * NUM_CORES + lax.axis_index("c")
  base = wid * ROWS_PER_WORKER

  dbufs = (d0, d1)
  cbufs = (c0, c1)
  dsems = (sem_d0, sem_d1)
  csems = (sem_c0, sem_c1)

  H = CHUNK // 2

  def issue(g):
    b = g % 2
    cp_d = pltpu.async_copy(
        data_hbm.at[pl.ds(base + g * CHUNK, CHUNK)], dbufs[b], dsems[b])
    cp_c0 = pltpu.async_copy(
        cen_hbm.at[idx_v.at[pl.ds(g * CHUNK, H)]],
        cbufs[b].at[pl.ds(0, H)], csems[b])
    cp_c1 = pltpu.async_copy(
        cen_hbm.at[idx_v.at[pl.ds(g * CHUNK + H, H)]],
        cbufs[b].at[pl.ds(H, H)], csems[b])
    return cp_d, cp_c0, cp_c1

  cp_i = pltpu.async_copy(labels_hbm.at[pl.ds(base, ROWS_PER_WORKER)], idx_v,
                          sem_i)
  cp_d0 = pltpu.async_copy(data_hbm.at[pl.ds(base, CHUNK)], d0, sem_d0)
  cp_d1 = pltpu.async_copy(data_hbm.at[pl.ds(base + CHUNK, CHUNK)], d1,
                           sem_d1)
  cp_i.wait()
  cp_c0a = pltpu.async_copy(cen_hbm.at[idx_v.at[pl.ds(0, H)]],
                            c0.at[pl.ds(0, H)], sem_c0)
  cp_c0b = pltpu.async_copy(cen_hbm.at[idx_v.at[pl.ds(H, H)]],
                            c0.at[pl.ds(H, H)], sem_c0)
  cp_c1a = pltpu.async_copy(cen_hbm.at[idx_v.at[pl.ds(CHUNK, H)]],
                            c1.at[pl.ds(0, H)], sem_c1)
  cp_c1b = pltpu.async_copy(cen_hbm.at[idx_v.at[pl.ds(CHUNK + H, H)]],
                            c1.at[pl.ds(H, H)], sem_c1)
  pend = {0: (cp_d0, cp_c0a, cp_c0b), 1: (cp_d1, cp_c1a, cp_c1b)}

  accs = tuple(jnp.zeros((LANES,), jnp.float32) for _ in range(VPR))

  for g in range(NUM_CHUNKS):
    cp_d, cp_c0, cp_c1 = pend.pop(g)
    cp_d.wait()
    cp_c0.wait()
    cp_c1.wait()
    dbuf = dbufs[g % 2]
    cbuf = cbufs[g % 2]

    def row_body(r, a, dbuf=dbuf, cbuf=cbuf):
      out = []
      for j in range(VPR):
        d = dbuf[r, pl.ds(j * LANES, LANES)]
        c = cbuf[r, pl.ds(j * LANES, LANES)]
        t = d - c
        out.append(a[j] + t * t)
      return tuple(out)

    accs = plsc.parallel_loop(0, CHUNK, 1, unroll=4, carry=accs)(row_body)
    if g + 2 < NUM_CHUNKS:
      pend[g + 2] = issue(g + 2)

  total = accs[0]
  for j in range(1, VPR):
    total = total + accs[j]
  acc_v[0] = total
  pltpu.sync_copy(acc_v, out_hbm.at[pl.ds(wid, 1)])


def _sc_partials(data, labels, cen):
  mesh = plsc.VectorSubcoreMesh(
      core_axis_name="c", subcore_axis_name="s",
      num_cores=NUM_CORES, num_subcores=NUM_SUBCORES)
  kern = pl.kernel(
      _sc_partial_body,
      out_type=jax.ShapeDtypeStruct((NUM_WORKERS, LANES), jnp.float32),
      mesh=mesh,
      scratch_types=[
          pltpu.VMEM((ROWS_PER_WORKER,), jnp.int32),
          pltpu.VMEM((CHUNK, DIM), jnp.float32),
          pltpu.VMEM((CHUNK, DIM), jnp.float32),
          pltpu.VMEM((CHUNK, DIM), jnp.float32),
          pltpu.VMEM((CHUNK, DIM), jnp.float32),
          pltpu.VMEM((1, LANES), jnp.float32),
          pltpu.SemaphoreType.DMA,
          pltpu.SemaphoreType.DMA,
          pltpu.SemaphoreType.DMA,
          pltpu.SemaphoreType.DMA,
          pltpu.SemaphoreType.DMA,
      ],
  )
  return kern(data, labels, cen)


def _tc_reduce_body(p_ref, o_ref):
  o_ref[0, 0] = jnp.sum(p_ref[...]) * (1.0 / BATCH)


def _tc_reduce(partials):
  return pl.pallas_call(
      _tc_reduce_body,
      out_shape=jax.ShapeDtypeStruct((1, 1), jnp.float32),
      in_specs=[pl.BlockSpec(memory_space=pltpu.VMEM)],
      out_specs=pl.BlockSpec(memory_space=pltpu.SMEM),
  )(partials)


@jax.jit
def kernel(data, labels, cen):
  partials = _sc_partials(data, labels.astype(jnp.int32), cen)
  return _tc_reduce(partials)[0, 0]


# speedup vs baseline: 1.0042x; 1.0042x over previous
import functools

import jax
import jax.numpy as jnp
from jax import lax
from jax.experimental import pallas as pl
from jax.experimental.pallas import tpu as pltpu
from jax.experimental.pallas import tpu_sc as plsc

BATCH = 16384
DIM = 128
LANES = 16
NUM_CORES = 2
NUM_SUBCORES = 16
NUM_WORKERS = NUM_CORES * NUM_SUBCORES
ROWS_PER_WORKER = BATCH // NUM_WORKERS
CHUNK = 128
NUM_CHUNKS = ROWS_PER_WORKER // CHUNK
VPR = DIM // LANES


def _sc_partial_body(data_hbm, labels_hbm, cen_hbm, out_hbm,
                     idx_v, d0, d1, c0, c1, acc_v,
                     sem_i, sem_d0, sem_d1, sem_c0, sem_c1):
  wid = lax.axis_index("s") * NUM_CORES + lax.axis_index("c")
  base = wid * ROWS_PER_WORKER

  pltpu.async_copy(labels_hbm.at[pl.ds(base, ROWS_PER_WORKER)], idx_v, sem_i)
  pltpu.async_copy(data_hbm.at[pl.ds(base, CHUNK)], d0, sem_d0)
  pltpu.async_copy(data_hbm.at[pl.ds(base + CHUNK, CHUNK)], d1, sem_d1)
  pltpu.make_async_copy(labels_hbm.at[pl.ds(base, ROWS_PER_WORKER)], idx_v,
                        sem_i).wait()
  pltpu.async_copy(cen_hbm.at[idx_v.at[pl.ds(0, CHUNK)]], c0, sem_c0)
  pltpu.async_copy(cen_hbm.at[idx_v.at[pl.ds(CHUNK, CHUNK)]], c1, sem_c1)

  def compute(dbuf, cbuf, accs):
    def row_body(r, a):
      out = []
      for j in range(VPR):
        d = dbuf[r, pl.ds(j * LANES, LANES)]
        c = cbuf[r, pl.ds(j * LANES, LANES)]
        t = d - c
        out.append(a[j] + t * t)
      return tuple(out)
    return plsc.parallel_loop(0, CHUNK, 1, unroll=4, carry=accs)(row_body)

  accs0 = tuple(jnp.zeros((LANES,), jnp.float32) for _ in range(VPR))

  def pair_body(i, accs):
    for (dbuf, cbuf, sem_d, sem_c, off) in (
        (d0, c0, sem_d0, sem_c0, 0),
        (d1, c1, sem_d1, sem_c1, 1)):
      g = 2 * i + off
      pltpu.make_async_copy(
          data_hbm.at[pl.ds(base + g * CHUNK, CHUNK)], dbuf, sem_d).wait()
      pltpu.make_async_copy(
          cen_hbm.at[idx_v.at[pl.ds(g * CHUNK, CHUNK)]], cbuf, sem_c).wait()
      accs = compute(dbuf, cbuf, accs)

      @pl.when(g + 2 < NUM_CHUNKS)
      def _():
        pltpu.async_copy(
            data_hbm.at[pl.ds(base + (g + 2) * CHUNK, CHUNK)], dbuf, sem_d)
        pltpu.async_copy(
            cen_hbm.at[idx_v.at[pl.ds((g + 2) * CHUNK, CHUNK)]], cbuf, sem_c)
    return accs

  accs = lax.fori_loop(0, NUM_CHUNKS // 2, pair_body, accs0)

  total = accs[0]
  for j in range(1, VPR):
    total = total + accs[j]
  acc_v[0] = total
  pltpu.sync_copy(acc_v, out_hbm.at[pl.ds(wid, 1)])


def _sc_partials(data, labels, cen):
  mesh = plsc.VectorSubcoreMesh(
      core_axis_name="c", subcore_axis_name="s",
      num_cores=NUM_CORES, num_subcores=NUM_SUBCORES)
  kern = pl.kernel(
      _sc_partial_body,
      out_type=jax.ShapeDtypeStruct((NUM_WORKERS, LANES), jnp.float32),
      mesh=mesh,
      scratch_types=[
          pltpu.VMEM((ROWS_PER_WORKER,), jnp.int32),
          pltpu.VMEM((CHUNK, DIM), jnp.float32),
          pltpu.VMEM((CHUNK, DIM), jnp.float32),
          pltpu.VMEM((CHUNK, DIM), jnp.float32),
          pltpu.VMEM((CHUNK, DIM), jnp.float32),
          pltpu.VMEM((1, LANES), jnp.float32),
          pltpu.SemaphoreType.DMA,
          pltpu.SemaphoreType.DMA,
          pltpu.SemaphoreType.DMA,
          pltpu.SemaphoreType.DMA,
          pltpu.SemaphoreType.DMA,
      ],
  )
  return kern(data, labels, cen)


def _tc_reduce_body(p_ref, o_ref):
  o_ref[0, 0] = jnp.sum(p_ref[...]) * (1.0 / BATCH)


def _tc_reduce(partials):
  return pl.pallas_call(
      _tc_reduce_body,
      out_shape=jax.ShapeDtypeStruct((1, 1), jnp.float32),
      in_specs=[pl.BlockSpec(memory_space=pltpu.VMEM)],
      out_specs=pl.BlockSpec(memory_space=pltpu.SMEM),
  )(partials)


@jax.jit
def kernel(data, labels, cen):
  partials = _sc_partials(data, labels.astype(jnp.int32), cen)
  return _tc_reduce(partials)[0, 0]
